# Initial kernel scaffold; baseline (speedup 1.0000x reference)
import jax
import jax.numpy as jnp
from jax.experimental import pallas as pl


def kernel(x, edge_index, batch, W1a, b1a, g1, be1, rm1, rv1, W1b, b1b, W2a, b2a, g2, be2, rm2, rv2, W2b, b2b, W3a, b3a, W3b, b3b):
    raise NotImplementedError("write your pallas kernel here")


# speedup vs baseline: 3.3424x; 3.3424x over previous
import functools

import jax
import jax.numpy as jnp
from jax import lax
from jax.experimental import pallas as pl
from jax.experimental.pallas import tpu as pltpu
from jax.experimental.pallas import tpu_sc as plsc

N = 10000
DIN = 128
F = 256
G = 16
E = 320000

NC = 2
NS = 16
NW = NC * NS
CH = 128
K = -(-E // (NW * CH))
EPAD = NW * CH * K
ACC_ROWS = 10112
RPT = ACC_ROWS // NS

RB = 2000
NBLK = N // RB

_sc_mesh = plsc.VectorSubcoreMesh(core_axis_name="c", subcore_axis_name="s")


@functools.partial(
    pl.kernel,
    mesh=_sc_mesh,
    out_type=jax.ShapeDtypeStruct((NC, ACC_ROWS, DIN), jnp.float32),
    scratch_types=[
        pltpu.VMEM((K, CH), jnp.int32),
        pltpu.VMEM((K, CH), jnp.int32),
        pltpu.VMEM((CH, DIN), jnp.float32),
        pltpu.VMEM_SHARED((ACC_ROWS, DIN), jnp.float32),
        pltpu.SemaphoreType.DMA,
    ],
)
def _sc_segment_sum(table, srcs, dsts, zeros, out, src_v, dst_v, rows_v, acc, sem):
    c = lax.axis_index("c")
    s = lax.axis_index("s")
    w = c * NS + s
    pltpu.sync_copy(zeros.at[pl.ds(s * RPT, RPT)], acc.at[pl.ds(s * RPT, RPT)])
    pltpu.sync_copy(srcs.at[w], src_v)
    pltpu.sync_copy(dsts.at[w], dst_v)
    plsc.subcore_barrier()

    def body(j, carry):
        pltpu.async_copy(table.at[src_v.at[j]], rows_v, sem).wait()
        pltpu.sync_copy(rows_v, acc.at[dst_v.at[j]], add=True)
        return carry

    lax.fori_loop(0, K, body, 0)
    plsc.subcore_barrier()
    pltpu.sync_copy(acc.at[pl.ds(s * RPT, RPT)], out.at[c, pl.ds(s * RPT, RPT)])


def _mlp1_body(x_ref, p0_ref, p1_ref, wa_ref, ba_ref, sc_ref, sh_ref, wb_ref,
               bb_ref, lo_ref, hi_ref):
    h = x_ref[...] + p0_ref[...] + p1_ref[...]
    h = jnp.dot(h, wa_ref[...], preferred_element_type=jnp.float32) + ba_ref[...]
    h = jnp.maximum(h * sc_ref[...] + sh_ref[...], 0.0)
    h = jnp.dot(h, wb_ref[...], preferred_element_type=jnp.float32) + bb_ref[...]
    h = jnp.maximum(h, 0.0)
    lo_ref[...] = h[:, :DIN]
    hi_ref[...] = h[:, DIN:]


def _mlp1(x, p0, p1, wa, ba, sc, sh, wb, bb):
    row = lambda i: (i, 0)
    full = lambda i: (0, 0)
    return pl.pallas_call(
        _mlp1_body,
        grid=(NBLK,),
        in_specs=[
            pl.BlockSpec((RB, DIN), row),
            pl.BlockSpec((RB, DIN), row),
            pl.BlockSpec((RB, DIN), row),
            pl.BlockSpec((DIN, F), full),
            pl.BlockSpec((1, F), full),
            pl.BlockSpec((1, F), full),
            pl.BlockSpec((1, F), full),
            pl.BlockSpec((F, F), full),
            pl.BlockSpec((1, F), full),
        ],
        out_specs=[pl.BlockSpec((RB, DIN), row), pl.BlockSpec((RB, DIN), row)],
        out_shape=[jax.ShapeDtypeStruct((N, DIN), jnp.float32)] * 2,
    )(x, p0, p1, wa, ba, sc, sh, wb, bb)


def _mlp2_body(lo_ref, hi_ref, q0_ref, q1_ref, q2_ref, q3_ref, bc_ref,
               wa_ref, ba_ref, sc_ref, sh_ref, wb_ref, bb_ref, w3a_ref, b3a_ref,
               w3b_ref, b3b_ref, out_ref, acc_sum, acc_max, acc_cnt):
    i = pl.program_id(0)
    hlo = lo_ref[...] + q0_ref[...] + q1_ref[...]
    hhi = hi_ref[...] + q2_ref[...] + q3_ref[...]
    h = jnp.concatenate([hlo, hhi], axis=1)
    h = jnp.dot(h, wa_ref[...], preferred_element_type=jnp.float32) + ba_ref[...]
    h = jnp.maximum(h * sc_ref[...] + sh_ref[...], 0.0)
    h = jnp.dot(h, wb_ref[...], preferred_element_type=jnp.float32) + bb_ref[...]
    h = jnp.maximum(h, 0.0)

    bc = bc_ref[...]
    gids = lax.broadcasted_iota(jnp.int32, (RB, G), 1)
    onehot = (gids == bc).astype(jnp.float32)
    dn = (((0,), (0,)), ((), ()))
    blk_sum = lax.dot_general(onehot, h, dn, preferred_element_type=jnp.float32)
    blk_cnt = lax.dot_general(onehot, jnp.ones_like(h), dn,
                              preferred_element_type=jnp.float32)
    neg = jnp.float32(-jnp.inf)
    blk_max = jnp.concatenate(
        [jnp.max(jnp.where(bc == g, h, neg), axis=0, keepdims=True)
         for g in range(G)], axis=0)

    @pl.when(i == 0)
    def _():
        acc_sum[...] = blk_sum
        acc_cnt[...] = blk_cnt
        acc_max[...] = blk_max

    @pl.when(i > 0)
    def _():
        acc_sum[...] += blk_sum
        acc_cnt[...] += blk_cnt
        acc_max[...] = jnp.maximum(acc_max[...], blk_max)

    @pl.when(i == NBLK - 1)
    def _():
        mean = acc_sum[...] / jnp.maximum(acc_cnt[...], 1.0)
        r = jnp.concatenate([mean, acc_max[...]], axis=1)
        t = jnp.maximum(
            jnp.dot(r, w3a_ref[...], preferred_element_type=jnp.float32)
            + b3a_ref[...], 0.0)
        out_ref[...] = (jnp.dot(t, w3b_ref[...], preferred_element_type=jnp.float32)
                        + b3b_ref[...])


def _mlp2(lo, hi, q0, q1, q2, q3, bc, wa, ba, sc, sh, wb, bb, w3a, b3a,
          w3b, b3b):
    row = lambda i: (i, 0)
    full = lambda i: (0, 0)
    return pl.pallas_call(
        _mlp2_body,
        grid=(NBLK,),
        in_specs=[
            pl.BlockSpec((RB, DIN), row),
            pl.BlockSpec((RB, DIN), row),
            pl.BlockSpec((RB, DIN), row),
            pl.BlockSpec((RB, DIN), row),
            pl.BlockSpec((RB, DIN), row),
            pl.BlockSpec((RB, DIN), row),
            pl.BlockSpec((RB, 1), row),
            pl.BlockSpec((F, F), full),
            pl.BlockSpec((1, F), full),
            pl.BlockSpec((1, F), full),
            pl.BlockSpec((1, F), full),
            pl.BlockSpec((F, F), full),
            pl.BlockSpec((1, F), full),
            pl.BlockSpec((2 * F, 4 * F), full),
            pl.BlockSpec((1, 4 * F), full),
            pl.BlockSpec((4 * F, 2), full),
            pl.BlockSpec((1, 2), full),
        ],
        out_specs=pl.BlockSpec((G, 2), full),
        out_shape=jax.ShapeDtypeStruct((G, 2), jnp.float32),
        scratch_shapes=[
            pltpu.VMEM((G, F), jnp.float32),
            pltpu.VMEM((G, F), jnp.float32),
            pltpu.VMEM((G, F), jnp.float32),
        ],
    )(lo, hi, q0, q1, q2, q3, bc, wa, ba, sc, sh, wb, bb, w3a, b3a, w3b, b3b)


def kernel(x, edge_index, batch, W1a, b1a, g1, be1, rm1, rv1, W1b, b1b,
           W2a, b2a, g2, be2, rm2, rv2, W2b, b2b, W3a, b3a, W3b, b3b):
    src = edge_index[0]
    dst = edge_index[1]
    pad = EPAD - E
    src_p = jnp.concatenate([src, jnp.zeros((pad,), jnp.int32)]).reshape(NW, K, CH)
    dst_p = jnp.concatenate([dst, jnp.full((pad,), N, jnp.int32)]).reshape(NW, K, CH)
    zeros = jnp.zeros((ACC_ROWS, DIN), jnp.float32)

    inv1 = g1 / jnp.sqrt(rv1 + 1e-5)
    sc1 = inv1.reshape(1, F)
    sh1 = (be1 - rm1 * inv1).reshape(1, F)
    inv2 = g2 / jnp.sqrt(rv2 + 1e-5)
    sc2 = inv2.reshape(1, F)
    sh2 = (be2 - rm2 * inv2).reshape(1, F)

    parts1 = _sc_segment_sum(x, src_p, dst_p, zeros)
    lo, hi = _mlp1(x, parts1[0, :N], parts1[1, :N], W1a, b1a.reshape(1, F),
                   sc1, sh1, W1b, b1b.reshape(1, F))

    parts_lo = _sc_segment_sum(lo, src_p, dst_p, zeros)
    parts_hi = _sc_segment_sum(hi, src_p, dst_p, zeros)

    bc = batch.reshape(N, 1)
    out = _mlp2(lo, hi, parts_lo[0, :N], parts_lo[1, :N], parts_hi[0, :N],
                parts_hi[1, :N], bc, W2a, b2a.reshape(1, F), sc2, sh2,
                W2b, b2b.reshape(1, F), W3a, b3a.reshape(1, 4 * F),
                W3b, b3b.reshape(1, 2))
    return out


# speedup vs baseline: 3.7495x; 1.1218x over previous
import functools

import jax
import jax.numpy as jnp
from jax import lax
from jax.experimental import pallas as pl
from jax.experimental.pallas import tpu as pltpu
from jax.experimental.pallas import tpu_sc as plsc

N = 10000
DIN = 128
F = 256
G = 16
E = 320000

NC = 2
NS = 16
NW = NC * NS
CH = 128
K = -(-E // (NW * CH))
EPAD = NW * CH * K
ACC_ROWS = 10112
RPT = ACC_ROWS // NS

RB = 2000
NBLK = N // RB

_sc_mesh = plsc.VectorSubcoreMesh(core_axis_name="c", subcore_axis_name="s")


HK = 40


@functools.partial(
    pl.kernel,
    mesh=_sc_mesh,
    out_type=jax.ShapeDtypeStruct((NC, ACC_ROWS, DIN), jnp.float32),
    scratch_types=[
        pltpu.VMEM((HK, CH), jnp.int32),
        pltpu.VMEM((HK, CH), jnp.int32),
        pltpu.VMEM((2, CH, DIN), jnp.float32),
        pltpu.VMEM_SHARED((ACC_ROWS, DIN), jnp.float32),
        pltpu.SemaphoreType.DMA((2,)),
        pltpu.SemaphoreType.DMA((2,)),
    ],
)
def _sc_segment_sum(table, srcs, dsts, zeros, out, src_v, dst_v, rows_v, acc,
                    gsem, ssem):
    c = lax.axis_index("c")
    s = lax.axis_index("s")
    w = c * NS + s
    pltpu.sync_copy(zeros.at[pl.ds(s * RPT, RPT)], acc.at[pl.ds(s * RPT, RPT)])
    plsc.subcore_barrier()

    def gather_start(j, b):
        pltpu.async_copy(table.at[src_v.at[j]], rows_v.at[b], gsem.at[b])

    def gather_wait(b):
        pltpu.make_async_copy(table.at[pl.ds(0, CH)], rows_v.at[b],
                              gsem.at[b]).wait()

    def scatter_start(j, b):
        pltpu.async_copy(rows_v.at[b], acc.at[dst_v.at[j]], ssem.at[b],
                         add=True)

    def scatter_wait(b):
        pltpu.make_async_copy(table.at[pl.ds(0, CH)], rows_v.at[b],
                              ssem.at[b]).wait()

    for p0, n in ((0, HK), (HK, K - HK)):
        pltpu.sync_copy(srcs.at[w, pl.ds(p0, n)], src_v.at[pl.ds(0, n)])
        pltpu.sync_copy(dsts.at[w, pl.ds(p0, n)], dst_v.at[pl.ds(0, n)])
        gather_start(0, 0)

        def pair(g, carry, n=n):
            for b in range(2):
                j = 2 * g + b

                @pl.when(j < n)
                def _():
                    gather_wait(b)
                    scatter_start(j, b)

                @pl.when(jnp.logical_and(j >= 1, j + 1 < n))
                def _():
                    scatter_wait(1 - b)

                @pl.when(j + 1 < n)
                def _():
                    gather_start(j + 1, 1 - b)
            return carry

        lax.fori_loop(0, -(-n // 2), pair, 0)
        scatter_wait((n - 2) % 2)
        scatter_wait((n - 1) % 2)

    plsc.subcore_barrier()
    pltpu.sync_copy(acc.at[pl.ds(s * RPT, RPT)], out.at[c, pl.ds(s * RPT, RPT)])


def _mlp1_body(x_ref, p0_ref, p1_ref, wa_ref, ba_ref, sc_ref, sh_ref, wb_ref,
               bb_ref, lo_ref, hi_ref):
    h = x_ref[...] + p0_ref[...] + p1_ref[...]
    h = jnp.dot(h, wa_ref[...], preferred_element_type=jnp.float32) + ba_ref[...]
    h = jnp.maximum(h * sc_ref[...] + sh_ref[...], 0.0)
    h = jnp.dot(h, wb_ref[...], preferred_element_type=jnp.float32) + bb_ref[...]
    h = jnp.maximum(h, 0.0)
    lo_ref[...] = h[:, :DIN]
    hi_ref[...] = h[:, DIN:]


def _mlp1(x, p0, p1, wa, ba, sc, sh, wb, bb):
    row = lambda i: (i, 0)
    full = lambda i: (0, 0)
    return pl.pallas_call(
        _mlp1_body,
        grid=(NBLK,),
        in_specs=[
            pl.BlockSpec((RB, DIN), row),
            pl.BlockSpec((RB, DIN), row),
            pl.BlockSpec((RB, DIN), row),
            pl.BlockSpec((DIN, F), full),
            pl.BlockSpec((1, F), full),
            pl.BlockSpec((1, F), full),
            pl.BlockSpec((1, F), full),
            pl.BlockSpec((F, F), full),
            pl.BlockSpec((1, F), full),
        ],
        out_specs=[pl.BlockSpec((RB, DIN), row), pl.BlockSpec((RB, DIN), row)],
        out_shape=[jax.ShapeDtypeStruct((N, DIN), jnp.float32)] * 2,
    )(x, p0, p1, wa, ba, sc, sh, wb, bb)


def _mlp2_body(lo_ref, hi_ref, q0_ref, q1_ref, q2_ref, q3_ref, bc_ref,
               wa_ref, ba_ref, sc_ref, sh_ref, wb_ref, bb_ref, w3a_ref, b3a_ref,
               w3b_ref, b3b_ref, out_ref, acc_sum, acc_max, acc_cnt):
    i = pl.program_id(0)
    hlo = lo_ref[...] + q0_ref[...] + q1_ref[...]
    hhi = hi_ref[...] + q2_ref[...] + q3_ref[...]
    h = jnp.concatenate([hlo, hhi], axis=1)
    h = jnp.dot(h, wa_ref[...], preferred_element_type=jnp.float32) + ba_ref[...]
    h = jnp.maximum(h * sc_ref[...] + sh_ref[...], 0.0)
    h = jnp.dot(h, wb_ref[...], preferred_element_type=jnp.float32) + bb_ref[...]
    h = jnp.maximum(h, 0.0)

    bc = bc_ref[...]
    gids = lax.broadcasted_iota(jnp.int32, (RB, G), 1)
    onehot = (gids == bc).astype(jnp.float32)
    dn = (((0,), (0,)), ((), ()))
    blk_sum = lax.dot_general(onehot, h, dn, preferred_element_type=jnp.float32)
    blk_cnt = lax.dot_general(onehot, jnp.ones_like(h), dn,
                              preferred_element_type=jnp.float32)
    neg = jnp.float32(-jnp.inf)
    blk_max = jnp.concatenate(
        [jnp.max(jnp.where(bc == g, h, neg), axis=0, keepdims=True)
         for g in range(G)], axis=0)

    @pl.when(i == 0)
    def _():
        acc_sum[...] = blk_sum
        acc_cnt[...] = blk_cnt
        acc_max[...] = blk_max

    @pl.when(i > 0)
    def _():
        acc_sum[...] += blk_sum
        acc_cnt[...] += blk_cnt
        acc_max[...] = jnp.maximum(acc_max[...], blk_max)

    @pl.when(i == NBLK - 1)
    def _():
        mean = acc_sum[...] / jnp.maximum(acc_cnt[...], 1.0)
        r = jnp.concatenate([mean, acc_max[...]], axis=1)
        t = jnp.maximum(
            jnp.dot(r, w3a_ref[...], preferred_element_type=jnp.float32)
            + b3a_ref[...], 0.0)
        out_ref[...] = (jnp.dot(t, w3b_ref[...], preferred_element_type=jnp.float32)
                        + b3b_ref[...])


def _mlp2(lo, hi, q0, q1, q2, q3, bc, wa, ba, sc, sh, wb, bb, w3a, b3a,
          w3b, b3b):
    row = lambda i: (i, 0)
    full = lambda i: (0, 0)
    return pl.pallas_call(
        _mlp2_body,
        grid=(NBLK,),
        in_specs=[
            pl.BlockSpec((RB, DIN), row),
            pl.BlockSpec((RB, DIN), row),
            pl.BlockSpec((RB, DIN), row),
            pl.BlockSpec((RB, DIN), row),
            pl.BlockSpec((RB, DIN), row),
            pl.BlockSpec((RB, DIN), row),
            pl.BlockSpec((RB, 1), row),
            pl.BlockSpec((F, F), full),
            pl.BlockSpec((1, F), full),
            pl.BlockSpec((1, F), full),
            pl.BlockSpec((1, F), full),
            pl.BlockSpec((F, F), full),
            pl.BlockSpec((1, F), full),
            pl.BlockSpec((2 * F, 4 * F), full),
            pl.BlockSpec((1, 4 * F), full),
            pl.BlockSpec((4 * F, 2), full),
            pl.BlockSpec((1, 2), full),
        ],
        out_specs=pl.BlockSpec((G, 2), full),
        out_shape=jax.ShapeDtypeStruct((G, 2), jnp.float32),
        scratch_shapes=[
            pltpu.VMEM((G, F), jnp.float32),
            pltpu.VMEM((G, F), jnp.float32),
            pltpu.VMEM((G, F), jnp.float32),
        ],
    )(lo, hi, q0, q1, q2, q3, bc, wa, ba, sc, sh, wb, bb, w3a, b3a, w3b, b3b)


def kernel(x, edge_index, batch, W1a, b1a, g1, be1, rm1, rv1, W1b, b1b,
           W2a, b2a, g2, be2, rm2, rv2, W2b, b2b, W3a, b3a, W3b, b3b):
    src = edge_index[0]
    dst = edge_index[1]
    pad = EPAD - E
    src_p = jnp.concatenate([src, jnp.zeros((pad,), jnp.int32)]).reshape(NW, K, CH)
    dst_p = jnp.concatenate([dst, jnp.full((pad,), N, jnp.int32)]).reshape(NW, K, CH)
    zeros = jnp.zeros((ACC_ROWS, DIN), jnp.float32)

    inv1 = g1 / jnp.sqrt(rv1 + 1e-5)
    sc1 = inv1.reshape(1, F)
    sh1 = (be1 - rm1 * inv1).reshape(1, F)
    inv2 = g2 / jnp.sqrt(rv2 + 1e-5)
    sc2 = inv2.reshape(1, F)
    sh2 = (be2 - rm2 * inv2).reshape(1, F)

    parts1 = _sc_segment_sum(x, src_p, dst_p, zeros)
    lo, hi = _mlp1(x, parts1[0, :N], parts1[1, :N], W1a, b1a.reshape(1, F),
                   sc1, sh1, W1b, b1b.reshape(1, F))

    parts_lo = _sc_segment_sum(lo, src_p, dst_p, zeros)
    parts_hi = _sc_segment_sum(hi, src_p, dst_p, zeros)

    bc = batch.reshape(N, 1)
    out = _mlp2(lo, hi, parts_lo[0, :N], parts_lo[1, :N], parts_hi[0, :N],
                parts_hi[1, :N], bc, W2a, b2a.reshape(1, F), sc2, sh2,
                W2b, b2b.reshape(1, F), W3a, b3a.reshape(1, 4 * F),
                W3b, b3b.reshape(1, 2))
    return out


# speedup vs baseline: 7.7667x; 2.0714x over previous
import functools

import jax
import jax.numpy as jnp
from jax import lax
from jax.experimental import pallas as pl
from jax.experimental.pallas import tpu as pltpu
from jax.experimental.pallas import tpu_sc as plsc

N = 10000
DIN = 128
F = 256
G = 16
E = 320000

NC = 2
NS = 16
NW = NC * NS
CH = 128
K = -(-E // (NW * CH))
EPAD = NW * CH * K
ACC_ROWS = 10112
RPT = ACC_ROWS // NS

RB = 2000
NBLK = N // RB

_sc_mesh = plsc.VectorSubcoreMesh(core_axis_name="c", subcore_axis_name="s")


HK = 40


@functools.partial(
    pl.kernel,
    mesh=_sc_mesh,
    out_type=jax.ShapeDtypeStruct((NC, ACC_ROWS, DIN), jnp.float32),
    scratch_types=[
        pltpu.VMEM((HK, CH), jnp.int32),
        pltpu.VMEM((HK, CH), jnp.int32),
        pltpu.VMEM((2, CH, DIN), jnp.float32),
        pltpu.VMEM_SHARED((ACC_ROWS, DIN), jnp.float32),
        pltpu.SemaphoreType.DMA((2,)),
        pltpu.SemaphoreType.DMA((2,)),
    ],
)
def _sc_segment_sum(table, srcs, dsts, zeros, out, src_v, dst_v, rows_v, acc,
                    gsem, ssem):
    c = lax.axis_index("c")
    s = lax.axis_index("s")
    w = c * NS + s
    pltpu.sync_copy(zeros.at[pl.ds(s * RPT, RPT)], acc.at[pl.ds(s * RPT, RPT)])
    plsc.subcore_barrier()

    def gather_start(j, b):
        pltpu.async_copy(table.at[src_v.at[j]], rows_v.at[b], gsem.at[b])

    def gather_wait(b):
        pltpu.make_async_copy(table.at[pl.ds(0, CH)], rows_v.at[b],
                              gsem.at[b]).wait()

    def scatter_start(j, b):
        pltpu.async_copy(rows_v.at[b], acc.at[dst_v.at[j]], ssem.at[b],
                         add=True)

    def scatter_wait(b):
        pltpu.make_async_copy(table.at[pl.ds(0, CH)], rows_v.at[b],
                              ssem.at[b]).wait()

    for p0, n in ((0, HK), (HK, K - HK)):
        pltpu.sync_copy(srcs.at[w, pl.ds(p0, n)], src_v.at[pl.ds(0, n)])
        pltpu.sync_copy(dsts.at[w, pl.ds(p0, n)], dst_v.at[pl.ds(0, n)])
        gather_start(0, 0)

        def pair(g, carry, n=n):
            for b in range(2):
                j = 2 * g + b

                @pl.when(j < n)
                def _():
                    gather_wait(b)
                    scatter_start(j, b)

                @pl.when(jnp.logical_and(j >= 1, j + 1 < n))
                def _():
                    scatter_wait(1 - b)

                @pl.when(j + 1 < n)
                def _():
                    gather_start(j + 1, 1 - b)
            return carry

        lax.fori_loop(0, -(-n // 2), pair, 0)
        scatter_wait((n - 2) % 2)
        scatter_wait((n - 1) % 2)

    plsc.subcore_barrier()
    pltpu.sync_copy(acc.at[pl.ds(s * RPT, RPT)], out.at[c, pl.ds(s * RPT, RPT)])


def _mlp1_body(x_ref, p0_ref, p1_ref, wa_ref, ba_ref, sc_ref, sh_ref, wb_ref,
               bb_ref, lo_ref, hi_ref):
    h = x_ref[...] + p0_ref[...] + p1_ref[...]
    h = jnp.dot(h, wa_ref[...], preferred_element_type=jnp.float32) + ba_ref[...]
    h = jnp.maximum(h * sc_ref[...] + sh_ref[...], 0.0)
    h = jnp.dot(h, wb_ref[...], preferred_element_type=jnp.float32) + bb_ref[...]
    h = jnp.maximum(h, 0.0)
    lo_ref[...] = h[:, :DIN]
    hi_ref[...] = h[:, DIN:]


def _mlp1(x, p0, p1, wa, ba, sc, sh, wb, bb):
    row = lambda i: (i, 0)
    full = lambda i: (0, 0)
    return pl.pallas_call(
        _mlp1_body,
        grid=(NBLK,),
        in_specs=[
            pl.BlockSpec((RB, DIN), row),
            pl.BlockSpec((RB, DIN), row),
            pl.BlockSpec((RB, DIN), row),
            pl.BlockSpec((DIN, F), full),
            pl.BlockSpec((1, F), full),
            pl.BlockSpec((1, F), full),
            pl.BlockSpec((1, F), full),
            pl.BlockSpec((F, F), full),
            pl.BlockSpec((1, F), full),
        ],
        out_specs=[pl.BlockSpec((RB, DIN), row), pl.BlockSpec((RB, DIN), row)],
        out_shape=[jax.ShapeDtypeStruct((N, DIN), jnp.float32)] * 2,
    )(x, p0, p1, wa, ba, sc, sh, wb, bb)


def _mlp2_body(lo_ref, hi_ref, q0_ref, q1_ref, q2_ref, q3_ref, bc_ref,
               wa_ref, ba_ref, sc_ref, sh_ref, wb_ref, bb_ref, w3a_ref, b3a_ref,
               w3b_ref, b3b_ref, out_ref, acc_sum, acc_max, acc_cnt):
    i = pl.program_id(0)
    hlo = lo_ref[...] + q0_ref[...] + q1_ref[...]
    hhi = hi_ref[...] + q2_ref[...] + q3_ref[...]
    h = jnp.concatenate([hlo, hhi], axis=1)
    h = jnp.dot(h, wa_ref[...], preferred_element_type=jnp.float32) + ba_ref[...]
    h = jnp.maximum(h * sc_ref[...] + sh_ref[...], 0.0)
    h = jnp.dot(h, wb_ref[...], preferred_element_type=jnp.float32) + bb_ref[...]
    h = jnp.maximum(h, 0.0)

    bc = bc_ref[...]
    gids = lax.broadcasted_iota(jnp.int32, (RB, G), 1)
    onehot = (gids == bc).astype(jnp.float32)
    dn = (((0,), (0,)), ((), ()))
    blk_sum = lax.dot_general(onehot, h, dn, preferred_element_type=jnp.float32)
    blk_cnt = lax.dot_general(onehot, jnp.ones_like(h), dn,
                              preferred_element_type=jnp.float32)
    neg = jnp.float32(-jnp.inf)
    blk_max = jnp.concatenate(
        [jnp.max(jnp.where(bc == g, h, neg), axis=0, keepdims=True)
         for g in range(G)], axis=0)

    @pl.when(i == 0)
    def _():
        acc_sum[...] = blk_sum
        acc_cnt[...] = blk_cnt
        acc_max[...] = blk_max

    @pl.when(i > 0)
    def _():
        acc_sum[...] += blk_sum
        acc_cnt[...] += blk_cnt
        acc_max[...] = jnp.maximum(acc_max[...], blk_max)

    @pl.when(i == NBLK - 1)
    def _():
        mean = acc_sum[...] / jnp.maximum(acc_cnt[...], 1.0)
        r = jnp.concatenate([mean, acc_max[...]], axis=1)
        t = jnp.maximum(
            jnp.dot(r, w3a_ref[...], preferred_element_type=jnp.float32)
            + b3a_ref[...], 0.0)
        out_ref[...] = (jnp.dot(t, w3b_ref[...], preferred_element_type=jnp.float32)
                        + b3b_ref[...])


def _mlp2(lo, hi, q0, q1, q2, q3, bc, wa, ba, sc, sh, wb, bb, w3a, b3a,
          w3b, b3b):
    row = lambda i: (i, 0)
    full = lambda i: (0, 0)
    return pl.pallas_call(
        _mlp2_body,
        grid=(NBLK,),
        in_specs=[
            pl.BlockSpec((RB, DIN), row),
            pl.BlockSpec((RB, DIN), row),
            pl.BlockSpec((RB, DIN), row),
            pl.BlockSpec((RB, DIN), row),
            pl.BlockSpec((RB, DIN), row),
            pl.BlockSpec((RB, DIN), row),
            pl.BlockSpec((RB, 1), row),
            pl.BlockSpec((F, F), full),
            pl.BlockSpec((1, F), full),
            pl.BlockSpec((1, F), full),
            pl.BlockSpec((1, F), full),
            pl.BlockSpec((F, F), full),
            pl.BlockSpec((1, F), full),
            pl.BlockSpec((2 * F, 4 * F), full),
            pl.BlockSpec((1, 4 * F), full),
            pl.BlockSpec((4 * F, 2), full),
            pl.BlockSpec((1, 2), full),
        ],
        out_specs=pl.BlockSpec((G, 2), full),
        out_shape=jax.ShapeDtypeStruct((G, 2), jnp.float32),
        scratch_shapes=[
            pltpu.VMEM((G, F), jnp.float32),
            pltpu.VMEM((G, F), jnp.float32),
            pltpu.VMEM((G, F), jnp.float32),
        ],
    )(lo, hi, q0, q1, q2, q3, bc, wa, ba, sc, sh, wb, bb, w3a, b3a, w3b, b3b)


def kernel(x, edge_index, batch, W1a, b1a, g1, be1, rm1, rv1, W1b, b1b,
           W2a, b2a, g2, be2, rm2, rv2, W2b, b2b, W3a, b3a, W3b, b3b):
    src = edge_index[0]
    dst = edge_index[1]
    pad = EPAD - E
    pad_ids = jnp.arange(pad, dtype=jnp.int32)
    src_p = jnp.concatenate([src, pad_ids % N]).reshape(NW, K, CH)
    dst_p = jnp.concatenate(
        [dst, N + pad_ids % (ACC_ROWS - N)]).reshape(NW, K, CH)
    zeros = jnp.zeros((ACC_ROWS, DIN), jnp.float32)

    inv1 = g1 / jnp.sqrt(rv1 + 1e-5)
    sc1 = inv1.reshape(1, F)
    sh1 = (be1 - rm1 * inv1).reshape(1, F)
    inv2 = g2 / jnp.sqrt(rv2 + 1e-5)
    sc2 = inv2.reshape(1, F)
    sh2 = (be2 - rm2 * inv2).reshape(1, F)

    parts1 = _sc_segment_sum(x, src_p, dst_p, zeros)
    lo, hi = _mlp1(x, parts1[0, :N], parts1[1, :N], W1a, b1a.reshape(1, F),
                   sc1, sh1, W1b, b1b.reshape(1, F))

    parts_lo = _sc_segment_sum(lo, src_p, dst_p, zeros)
    parts_hi = _sc_segment_sum(hi, src_p, dst_p, zeros)

    bc = batch.reshape(N, 1)
    out = _mlp2(lo, hi, parts_lo[0, :N], parts_lo[1, :N], parts_hi[0, :N],
                parts_hi[1, :N], bc, W2a, b2a.reshape(1, F), sc2, sh2,
                W2b, b2b.reshape(1, F), W3a, b3a.reshape(1, 4 * F),
                W3b, b3b.reshape(1, 2))
    return out


# speedup vs baseline: 8.1182x; 1.0453x over previous
import functools

import jax
import jax.numpy as jnp
from jax import lax
from jax.experimental import pallas as pl
from jax.experimental.pallas import tpu as pltpu
from jax.experimental.pallas import tpu_sc as plsc

N = 10000
DIN = 128
F = 256
G = 16
E = 320000

NC = 2
NS = 16
NW = NC * NS
CH = 128
K = -(-E // (NW * CH))
EPAD = NW * CH * K
ACC_ROWS = 10112
RPT = ACC_ROWS // NS

K2 = -(-E // (NS * CH))
EPAD2 = NS * CH * K2

RB = 2000
NBLK = N // RB

_sc_mesh = plsc.VectorSubcoreMesh(core_axis_name="c", subcore_axis_name="s")


HK = 40


_SC_SCRATCH = [
    pltpu.VMEM((HK, CH), jnp.int32),
    pltpu.VMEM((HK, CH), jnp.int32),
    pltpu.VMEM((2, CH, DIN), jnp.float32),
    pltpu.VMEM_SHARED((ACC_ROWS, DIN), jnp.float32),
    pltpu.SemaphoreType.DMA((2,)),
    pltpu.SemaphoreType.DMA((2,)),
]


def _edge_pipeline(table, srcs, dsts, w, nchunks, src_v, dst_v, rows_v, acc,
                   gsem, ssem):
    def gather_start(j, b):
        pltpu.async_copy(table.at[src_v.at[j]], rows_v.at[b], gsem.at[b])

    def gather_wait(b):
        pltpu.make_async_copy(table.at[pl.ds(0, CH)], rows_v.at[b],
                              gsem.at[b]).wait()

    def scatter_start(j, b):
        pltpu.async_copy(rows_v.at[b], acc.at[dst_v.at[j]], ssem.at[b],
                         add=True)

    def scatter_wait(b):
        pltpu.make_async_copy(table.at[pl.ds(0, CH)], rows_v.at[b],
                              ssem.at[b]).wait()

    for p0 in range(0, nchunks, HK):
        n = min(HK, nchunks - p0)
        pltpu.sync_copy(srcs.at[w, pl.ds(p0, n)], src_v.at[pl.ds(0, n)])
        pltpu.sync_copy(dsts.at[w, pl.ds(p0, n)], dst_v.at[pl.ds(0, n)])
        gather_start(0, 0)

        def pair(g, carry, n=n):
            for b in range(2):
                j = 2 * g + b

                @pl.when(j < n)
                def _():
                    gather_wait(b)
                    scatter_start(j, b)

                @pl.when(jnp.logical_and(j >= 1, j + 1 < n))
                def _():
                    scatter_wait(1 - b)

                @pl.when(j + 1 < n)
                def _():
                    gather_start(j + 1, 1 - b)
            return carry

        lax.fori_loop(0, -(-n // 2), pair, 0)
        scatter_wait((n - 2) % 2)
        scatter_wait((n - 1) % 2)


@functools.partial(
    pl.kernel,
    mesh=_sc_mesh,
    out_type=jax.ShapeDtypeStruct((NC, ACC_ROWS, DIN), jnp.float32),
    scratch_types=_SC_SCRATCH,
)
def _sc_segment_sum(table, srcs, dsts, zeros, out, src_v, dst_v, rows_v, acc,
                    gsem, ssem):
    c = lax.axis_index("c")
    s = lax.axis_index("s")
    w = c * NS + s
    pltpu.sync_copy(zeros.at[pl.ds(s * RPT, RPT)], acc.at[pl.ds(s * RPT, RPT)])
    plsc.subcore_barrier()
    _edge_pipeline(table, srcs, dsts, w, K, src_v, dst_v, rows_v, acc,
                   gsem, ssem)
    plsc.subcore_barrier()
    pltpu.sync_copy(acc.at[pl.ds(s * RPT, RPT)], out.at[c, pl.ds(s * RPT, RPT)])


@functools.partial(
    pl.kernel,
    mesh=_sc_mesh,
    out_type=jax.ShapeDtypeStruct((NC, ACC_ROWS, DIN), jnp.float32),
    scratch_types=_SC_SCRATCH,
)
def _sc_segment_sum2(lo, hi, srcs, dsts, zeros, out, src_v, dst_v, rows_v,
                     acc, gsem, ssem):
    c = lax.axis_index("c")
    s = lax.axis_index("s")
    pltpu.sync_copy(zeros.at[pl.ds(s * RPT, RPT)], acc.at[pl.ds(s * RPT, RPT)])
    plsc.subcore_barrier()

    @pl.when(c == 0)
    def _():
        _edge_pipeline(lo, srcs, dsts, s, K2, src_v, dst_v, rows_v, acc,
                       gsem, ssem)

    @pl.when(c == 1)
    def _():
        _edge_pipeline(hi, srcs, dsts, s, K2, src_v, dst_v, rows_v, acc,
                       gsem, ssem)

    plsc.subcore_barrier()
    pltpu.sync_copy(acc.at[pl.ds(s * RPT, RPT)], out.at[c, pl.ds(s * RPT, RPT)])


def _mlp1_body(x_ref, p0_ref, p1_ref, wa_ref, ba_ref, sc_ref, sh_ref, wb_ref,
               bb_ref, lo_ref, hi_ref):
    h = x_ref[...] + p0_ref[...] + p1_ref[...]
    h = jnp.dot(h, wa_ref[...], preferred_element_type=jnp.float32) + ba_ref[...]
    h = jnp.maximum(h * sc_ref[...] + sh_ref[...], 0.0)
    h = jnp.dot(h, wb_ref[...], preferred_element_type=jnp.float32) + bb_ref[...]
    h = jnp.maximum(h, 0.0)
    lo_ref[...] = h[:, :DIN]
    hi_ref[...] = h[:, DIN:]


def _mlp1(x, p0, p1, wa, ba, sc, sh, wb, bb):
    row = lambda i: (i, 0)
    full = lambda i: (0, 0)
    return pl.pallas_call(
        _mlp1_body,
        grid=(NBLK,),
        in_specs=[
            pl.BlockSpec((RB, DIN), row),
            pl.BlockSpec((RB, DIN), row),
            pl.BlockSpec((RB, DIN), row),
            pl.BlockSpec((DIN, F), full),
            pl.BlockSpec((1, F), full),
            pl.BlockSpec((1, F), full),
            pl.BlockSpec((1, F), full),
            pl.BlockSpec((F, F), full),
            pl.BlockSpec((1, F), full),
        ],
        out_specs=[pl.BlockSpec((RB, DIN), row), pl.BlockSpec((RB, DIN), row)],
        out_shape=[jax.ShapeDtypeStruct((N, DIN), jnp.float32)] * 2,
    )(x, p0, p1, wa, ba, sc, sh, wb, bb)


def _mlp2_body(lo_ref, hi_ref, q0_ref, q1_ref, bc_ref,
               wa_ref, ba_ref, sc_ref, sh_ref, wb_ref, bb_ref, w3a_ref, b3a_ref,
               w3b_ref, b3b_ref, out_ref, acc_sum, acc_max, acc_cnt):
    i = pl.program_id(0)
    hlo = lo_ref[...] + q0_ref[...]
    hhi = hi_ref[...] + q1_ref[...]
    h = jnp.concatenate([hlo, hhi], axis=1)
    h = jnp.dot(h, wa_ref[...], preferred_element_type=jnp.float32) + ba_ref[...]
    h = jnp.maximum(h * sc_ref[...] + sh_ref[...], 0.0)
    h = jnp.dot(h, wb_ref[...], preferred_element_type=jnp.float32) + bb_ref[...]
    h = jnp.maximum(h, 0.0)

    bc = bc_ref[...]
    gids = lax.broadcasted_iota(jnp.int32, (RB, G), 1)
    onehot = (gids == bc).astype(jnp.float32)
    dn = (((0,), (0,)), ((), ()))
    blk_sum = lax.dot_general(onehot, h, dn, preferred_element_type=jnp.float32)
    blk_cnt = lax.dot_general(onehot, jnp.ones_like(h), dn,
                              preferred_element_type=jnp.float32)
    neg = jnp.float32(-jnp.inf)
    blk_max = jnp.concatenate(
        [jnp.max(jnp.where(bc == g, h, neg), axis=0, keepdims=True)
         for g in range(G)], axis=0)

    @pl.when(i == 0)
    def _():
        acc_sum[...] = blk_sum
        acc_cnt[...] = blk_cnt
        acc_max[...] = blk_max

    @pl.when(i > 0)
    def _():
        acc_sum[...] += blk_sum
        acc_cnt[...] += blk_cnt
        acc_max[...] = jnp.maximum(acc_max[...], blk_max)

    @pl.when(i == NBLK - 1)
    def _():
        mean = acc_sum[...] / jnp.maximum(acc_cnt[...], 1.0)
        r = jnp.concatenate([mean, acc_max[...]], axis=1)
        t = jnp.maximum(
            jnp.dot(r, w3a_ref[...], preferred_element_type=jnp.float32)
            + b3a_ref[...], 0.0)
        out_ref[...] = (jnp.dot(t, w3b_ref[...], preferred_element_type=jnp.float32)
                        + b3b_ref[...])


def _mlp2(lo, hi, q0, q1, bc, wa, ba, sc, sh, wb, bb, w3a, b3a,
          w3b, b3b):
    row = lambda i: (i, 0)
    full = lambda i: (0, 0)
    return pl.pallas_call(
        _mlp2_body,
        grid=(NBLK,),
        in_specs=[
            pl.BlockSpec((RB, DIN), row),
            pl.BlockSpec((RB, DIN), row),
            pl.BlockSpec((RB, DIN), row),
            pl.BlockSpec((RB, DIN), row),
            pl.BlockSpec((RB, 1), row),
            pl.BlockSpec((F, F), full),
            pl.BlockSpec((1, F), full),
            pl.BlockSpec((1, F), full),
            pl.BlockSpec((1, F), full),
            pl.BlockSpec((F, F), full),
            pl.BlockSpec((1, F), full),
            pl.BlockSpec((2 * F, 4 * F), full),
            pl.BlockSpec((1, 4 * F), full),
            pl.BlockSpec((4 * F, 2), full),
            pl.BlockSpec((1, 2), full),
        ],
        out_specs=pl.BlockSpec((G, 2), full),
        out_shape=jax.ShapeDtypeStruct((G, 2), jnp.float32),
        scratch_shapes=[
            pltpu.VMEM((G, F), jnp.float32),
            pltpu.VMEM((G, F), jnp.float32),
            pltpu.VMEM((G, F), jnp.float32),
        ],
    )(lo, hi, q0, q1, bc, wa, ba, sc, sh, wb, bb, w3a, b3a, w3b, b3b)


def kernel(x, edge_index, batch, W1a, b1a, g1, be1, rm1, rv1, W1b, b1b,
           W2a, b2a, g2, be2, rm2, rv2, W2b, b2b, W3a, b3a, W3b, b3b):
    src = edge_index[0]
    dst = edge_index[1]
    pad = EPAD - E
    pad_ids = jnp.arange(pad, dtype=jnp.int32)
    src_p = jnp.concatenate([src, pad_ids % N]).reshape(NW, K, CH)
    dst_p = jnp.concatenate(
        [dst, N + pad_ids % (ACC_ROWS - N)]).reshape(NW, K, CH)
    zeros = jnp.zeros((ACC_ROWS, DIN), jnp.float32)

    inv1 = g1 / jnp.sqrt(rv1 + 1e-5)
    sc1 = inv1.reshape(1, F)
    sh1 = (be1 - rm1 * inv1).reshape(1, F)
    inv2 = g2 / jnp.sqrt(rv2 + 1e-5)
    sc2 = inv2.reshape(1, F)
    sh2 = (be2 - rm2 * inv2).reshape(1, F)

    pad2_ids = jnp.arange(EPAD2 - E, dtype=jnp.int32)
    src2 = jnp.concatenate([src, pad2_ids % N]).reshape(NS, K2, CH)
    dst2 = jnp.concatenate(
        [dst, N + pad2_ids % (ACC_ROWS - N)]).reshape(NS, K2, CH)

    parts1 = _sc_segment_sum(x, src_p, dst_p, zeros)
    lo, hi = _mlp1(x, parts1[0, :N], parts1[1, :N], W1a, b1a.reshape(1, F),
                   sc1, sh1, W1b, b1b.reshape(1, F))

    parts2 = _sc_segment_sum2(lo, hi, src2, dst2, zeros)

    bc = batch.reshape(N, 1)
    out = _mlp2(lo, hi, parts2[0, :N], parts2[1, :N], bc,
                W2a, b2a.reshape(1, F), sc2, sh2,
                W2b, b2b.reshape(1, F), W3a, b3a.reshape(1, 4 * F),
                W3b, b3b.reshape(1, 2))
    return out


# speedup vs baseline: 8.3520x; 1.0288x over previous
import functools

import jax
import jax.numpy as jnp
from jax import lax
from jax.experimental import pallas as pl
from jax.experimental.pallas import tpu as pltpu
from jax.experimental.pallas import tpu_sc as plsc

N = 10000
DIN = 128
F = 256
G = 16
E = 320000

NC = 2
NS = 16
NW = NC * NS
CH = 128
K = -(-E // (NW * CH))
EPAD = NW * CH * K
ACC_ROWS = 10112
RPT = ACC_ROWS // NS

K2 = -(-E // (NS * CH))
EPAD2 = NS * CH * K2

RB = 2000
NBLK = N // RB

_sc_mesh = plsc.VectorSubcoreMesh(core_axis_name="c", subcore_axis_name="s")


HK = 40


NBUF = 2
LOOK = 1

_SC_SCRATCH = [
    pltpu.VMEM((HK, CH), jnp.int32),
    pltpu.VMEM((HK, CH), jnp.int32),
    pltpu.VMEM((NBUF, CH, DIN), jnp.float32),
    pltpu.VMEM_SHARED((ACC_ROWS, DIN), jnp.float32),
    pltpu.SemaphoreType.DMA((NBUF,)),
    pltpu.SemaphoreType.DMA((NBUF,)),
]


def _edge_pipeline(table, srcs, dsts, w, nchunks, src_v, dst_v, rows_v, acc,
                   gsem, ssem):
    def gather_start(j, b):
        pltpu.async_copy(table.at[src_v.at[j]], rows_v.at[b], gsem.at[b])

    def gather_wait(b):
        pltpu.make_async_copy(table.at[pl.ds(0, CH)], rows_v.at[b],
                              gsem.at[b]).wait()

    def scatter_start(j, b):
        pltpu.async_copy(rows_v.at[b], acc.at[dst_v.at[j]], ssem.at[b],
                         add=True)

    def scatter_wait(b):
        pltpu.make_async_copy(table.at[pl.ds(0, CH)], rows_v.at[b],
                              ssem.at[b]).wait()

    for p0 in range(0, nchunks, HK):
        n = min(HK, nchunks - p0)
        pltpu.sync_copy(srcs.at[w, pl.ds(p0, n)], src_v.at[pl.ds(0, n)])
        pltpu.sync_copy(dsts.at[w, pl.ds(p0, n)], dst_v.at[pl.ds(0, n)])
        for i in range(LOOK):
            gather_start(i, i)

        def grp(g, carry, n=n):
            for b in range(NBUF):
                j = NBUF * g + b
                bb = (b + LOOK) % NBUF

                @pl.when(j < n)
                def _():
                    gather_wait(b)
                    scatter_start(j, b)

                @pl.when(jnp.logical_and(j >= NBUF - LOOK, j + LOOK < n))
                def _():
                    scatter_wait(bb)

                @pl.when(j + LOOK < n)
                def _():
                    gather_start(j + LOOK, bb)
            return carry

        lax.fori_loop(0, -(-n // NBUF), grp, 0)
        for b in range(NBUF):
            scatter_wait(b)


_SC_OUT = [jax.ShapeDtypeStruct((ACC_ROWS, DIN), jnp.float32)] * 2


def _core_out_copy(c, s, acc, out0, out1):
    @pl.when(c == 0)
    def _():
        pltpu.sync_copy(acc.at[pl.ds(s * RPT, RPT)],
                        out0.at[pl.ds(s * RPT, RPT)])

    @pl.when(c == 1)
    def _():
        pltpu.sync_copy(acc.at[pl.ds(s * RPT, RPT)],
                        out1.at[pl.ds(s * RPT, RPT)])


@functools.partial(
    pl.kernel,
    mesh=_sc_mesh,
    out_type=_SC_OUT,
    scratch_types=_SC_SCRATCH,
)
def _sc_segment_sum(table, srcs, dsts, zeros, out0, out1, src_v, dst_v,
                    rows_v, acc, gsem, ssem):
    c = lax.axis_index("c")
    s = lax.axis_index("s")
    w = c * NS + s
    pltpu.sync_copy(zeros.at[pl.ds(s * RPT, RPT)], acc.at[pl.ds(s * RPT, RPT)])
    plsc.subcore_barrier()
    _edge_pipeline(table, srcs, dsts, w, K, src_v, dst_v, rows_v, acc,
                   gsem, ssem)
    plsc.subcore_barrier()
    _core_out_copy(c, s, acc, out0, out1)


@functools.partial(
    pl.kernel,
    mesh=_sc_mesh,
    out_type=_SC_OUT,
    scratch_types=_SC_SCRATCH,
)
def _sc_segment_sum2(lo, hi, srcs, dsts, zeros, out0, out1, src_v, dst_v,
                     rows_v, acc, gsem, ssem):
    c = lax.axis_index("c")
    s = lax.axis_index("s")
    pltpu.sync_copy(zeros.at[pl.ds(s * RPT, RPT)], acc.at[pl.ds(s * RPT, RPT)])
    plsc.subcore_barrier()

    @pl.when(c == 0)
    def _():
        _edge_pipeline(lo, srcs, dsts, s, K2, src_v, dst_v, rows_v, acc,
                       gsem, ssem)

    @pl.when(c == 1)
    def _():
        _edge_pipeline(hi, srcs, dsts, s, K2, src_v, dst_v, rows_v, acc,
                       gsem, ssem)

    plsc.subcore_barrier()
    _core_out_copy(c, s, acc, out0, out1)


def _mlp1_body(x_ref, p0_ref, p1_ref, wa_ref, ba_ref, sc_ref, sh_ref, wb_ref,
               bb_ref, lo_ref, hi_ref):
    h = x_ref[...] + p0_ref[...] + p1_ref[...]
    h = jnp.dot(h, wa_ref[...], preferred_element_type=jnp.float32) + ba_ref[...]
    h = jnp.maximum(h * sc_ref[...] + sh_ref[...], 0.0)
    h = jnp.dot(h, wb_ref[...], preferred_element_type=jnp.float32) + bb_ref[...]
    h = jnp.maximum(h, 0.0)
    lo_ref[...] = h[:, :DIN]
    hi_ref[...] = h[:, DIN:]


def _mlp1(x, p0, p1, wa, ba, sc, sh, wb, bb):
    row = lambda i: (i, 0)
    full = lambda i: (0, 0)
    return pl.pallas_call(
        _mlp1_body,
        grid=(NBLK,),
        in_specs=[
            pl.BlockSpec((RB, DIN), row),
            pl.BlockSpec((RB, DIN), row),
            pl.BlockSpec((RB, DIN), row),
            pl.BlockSpec((DIN, F), full),
            pl.BlockSpec((1, F), full),
            pl.BlockSpec((1, F), full),
            pl.BlockSpec((1, F), full),
            pl.BlockSpec((F, F), full),
            pl.BlockSpec((1, F), full),
        ],
        out_specs=[pl.BlockSpec((RB, DIN), row), pl.BlockSpec((RB, DIN), row)],
        out_shape=[jax.ShapeDtypeStruct((N, DIN), jnp.float32)] * 2,
    )(x, p0, p1, wa, ba, sc, sh, wb, bb)


def _mlp2_body(lo_ref, hi_ref, q0_ref, q1_ref, bc_ref,
               wa_ref, ba_ref, sc_ref, sh_ref, wb_ref, bb_ref, w3a_ref, b3a_ref,
               w3b_ref, b3b_ref, out_ref, acc_sum, acc_max, acc_cnt):
    i = pl.program_id(0)
    hlo = lo_ref[...] + q0_ref[...]
    hhi = hi_ref[...] + q1_ref[...]
    h = jnp.concatenate([hlo, hhi], axis=1)
    h = jnp.dot(h, wa_ref[...], preferred_element_type=jnp.float32) + ba_ref[...]
    h = jnp.maximum(h * sc_ref[...] + sh_ref[...], 0.0)
    h = jnp.dot(h, wb_ref[...], preferred_element_type=jnp.float32) + bb_ref[...]
    h = jnp.maximum(h, 0.0)

    bc = bc_ref[...]
    gids = lax.broadcasted_iota(jnp.int32, (RB, G), 1)
    onehot = (gids == bc).astype(jnp.float32)
    dn = (((0,), (0,)), ((), ()))
    blk_sum = lax.dot_general(onehot, h, dn, preferred_element_type=jnp.float32)
    blk_cnt = lax.dot_general(onehot, jnp.ones_like(h), dn,
                              preferred_element_type=jnp.float32)
    neg = jnp.float32(-jnp.inf)
    blk_max = jnp.concatenate(
        [jnp.max(jnp.where(bc == g, h, neg), axis=0, keepdims=True)
         for g in range(G)], axis=0)

    @pl.when(i == 0)
    def _():
        acc_sum[...] = blk_sum
        acc_cnt[...] = blk_cnt
        acc_max[...] = blk_max

    @pl.when(i > 0)
    def _():
        acc_sum[...] += blk_sum
        acc_cnt[...] += blk_cnt
        acc_max[...] = jnp.maximum(acc_max[...], blk_max)

    @pl.when(i == NBLK - 1)
    def _():
        mean = acc_sum[...] / jnp.maximum(acc_cnt[...], 1.0)
        r = jnp.concatenate([mean, acc_max[...]], axis=1)
        t = jnp.maximum(
            jnp.dot(r, w3a_ref[...], preferred_element_type=jnp.float32)
            + b3a_ref[...], 0.0)
        out_ref[...] = (jnp.dot(t, w3b_ref[...], preferred_element_type=jnp.float32)
                        + b3b_ref[...])


def _mlp2(lo, hi, q0, q1, bc, wa, ba, sc, sh, wb, bb, w3a, b3a,
          w3b, b3b):
    row = lambda i: (i, 0)
    full = lambda i: (0, 0)
    return pl.pallas_call(
        _mlp2_body,
        grid=(NBLK,),
        in_specs=[
            pl.BlockSpec((RB, DIN), row),
            pl.BlockSpec((RB, DIN), row),
            pl.BlockSpec((RB, DIN), row),
            pl.BlockSpec((RB, DIN), row),
            pl.BlockSpec((RB, 1), row),
            pl.BlockSpec((F, F), full),
            pl.BlockSpec((1, F), full),
            pl.BlockSpec((1, F), full),
            pl.BlockSpec((1, F), full),
            pl.BlockSpec((F, F), full),
            pl.BlockSpec((1, F), full),
            pl.BlockSpec((2 * F, 4 * F), full),
            pl.BlockSpec((1, 4 * F), full),
            pl.BlockSpec((4 * F, 2), full),
            pl.BlockSpec((1, 2), full),
        ],
        out_specs=pl.BlockSpec((G, 2), full),
        out_shape=jax.ShapeDtypeStruct((G, 2), jnp.float32),
        scratch_shapes=[
            pltpu.VMEM((G, F), jnp.float32),
            pltpu.VMEM((G, F), jnp.float32),
            pltpu.VMEM((G, F), jnp.float32),
        ],
    )(lo, hi, q0, q1, bc, wa, ba, sc, sh, wb, bb, w3a, b3a, w3b, b3b)


def kernel(x, edge_index, batch, W1a, b1a, g1, be1, rm1, rv1, W1b, b1b,
           W2a, b2a, g2, be2, rm2, rv2, W2b, b2b, W3a, b3a, W3b, b3b):
    src = edge_index[0]
    dst = edge_index[1]
    pad = EPAD - E
    pad_ids = jnp.arange(pad, dtype=jnp.int32)
    src_p = jnp.concatenate([src, pad_ids % N]).reshape(NW, K, CH)
    dst_p = jnp.concatenate(
        [dst, N + pad_ids % (ACC_ROWS - N)]).reshape(NW, K, CH)
    zeros = jnp.zeros((ACC_ROWS, DIN), jnp.float32)

    inv1 = g1 / jnp.sqrt(rv1 + 1e-5)
    sc1 = inv1.reshape(1, F)
    sh1 = (be1 - rm1 * inv1).reshape(1, F)
    inv2 = g2 / jnp.sqrt(rv2 + 1e-5)
    sc2 = inv2.reshape(1, F)
    sh2 = (be2 - rm2 * inv2).reshape(1, F)

    pad2_ids = jnp.arange(EPAD2 - E, dtype=jnp.int32)
    src2 = jnp.concatenate([src, pad2_ids % N]).reshape(NS, K2, CH)
    dst2 = jnp.concatenate(
        [dst, N + pad2_ids % (ACC_ROWS - N)]).reshape(NS, K2, CH)

    p1a, p1b = _sc_segment_sum(x, src_p, dst_p, zeros)
    lo, hi = _mlp1(x, p1a, p1b, W1a, b1a.reshape(1, F),
                   sc1, sh1, W1b, b1b.reshape(1, F))

    agg_lo, agg_hi = _sc_segment_sum2(lo, hi, src2, dst2, zeros)

    bc = batch.reshape(N, 1)
    out = _mlp2(lo, hi, agg_lo, agg_hi, bc,
                W2a, b2a.reshape(1, F), sc2, sh2,
                W2b, b2b.reshape(1, F), W3a, b3a.reshape(1, 4 * F),
                W3b, b3b.reshape(1, 2))
    return out


# speedup vs baseline: 8.5422x; 1.0228x over previous
import functools

import jax
import jax.numpy as jnp
from jax import lax
from jax.experimental import pallas as pl
from jax.experimental.pallas import tpu as pltpu
from jax.experimental.pallas import tpu_sc as plsc

N = 10000
DIN = 128
F = 256
G = 16
E = 320000

NC = 2
NS = 16
NW = NC * NS
CH = 64
K = -(-E // (NW * CH))
EPAD = NW * CH * K
ACC_ROWS = 10112
RPT = ACC_ROWS // NS

K2 = -(-E // (NS * CH))
EPAD2 = NS * CH * K2

RB = 2000
NBLK = N // RB

_sc_mesh = plsc.VectorSubcoreMesh(core_axis_name="c", subcore_axis_name="s")


HK = 40


NBUF = 4
LOOK = 2

_SC_SCRATCH = [
    pltpu.VMEM((HK, CH), jnp.int32),
    pltpu.VMEM((HK, CH), jnp.int32),
    pltpu.VMEM((NBUF, CH, DIN), jnp.float32),
    pltpu.VMEM_SHARED((ACC_ROWS, DIN), jnp.float32),
    pltpu.SemaphoreType.DMA((NBUF,)),
    pltpu.SemaphoreType.DMA((NBUF,)),
]


def _edge_pipeline(table, srcs, dsts, w, nchunks, src_v, dst_v, rows_v, acc,
                   gsem, ssem):
    def gather_start(j, b):
        pltpu.async_copy(table.at[src_v.at[j]], rows_v.at[b], gsem.at[b])

    def gather_wait(b):
        pltpu.make_async_copy(table.at[pl.ds(0, CH)], rows_v.at[b],
                              gsem.at[b]).wait()

    def scatter_start(j, b):
        pltpu.async_copy(rows_v.at[b], acc.at[dst_v.at[j]], ssem.at[b],
                         add=True)

    def scatter_wait(b):
        pltpu.make_async_copy(table.at[pl.ds(0, CH)], rows_v.at[b],
                              ssem.at[b]).wait()

    for p0 in range(0, nchunks, HK):
        n = min(HK, nchunks - p0)
        pltpu.sync_copy(srcs.at[w, pl.ds(p0, n)], src_v.at[pl.ds(0, n)])
        pltpu.sync_copy(dsts.at[w, pl.ds(p0, n)], dst_v.at[pl.ds(0, n)])
        for i in range(LOOK):
            gather_start(i, i)

        def grp(g, carry, n=n):
            for b in range(NBUF):
                j = NBUF * g + b
                bb = (b + LOOK) % NBUF

                @pl.when(j < n)
                def _():
                    gather_wait(b)
                    scatter_start(j, b)

                @pl.when(jnp.logical_and(j >= NBUF - LOOK, j + LOOK < n))
                def _():
                    scatter_wait(bb)

                @pl.when(j + LOOK < n)
                def _():
                    gather_start(j + LOOK, bb)
            return carry

        lax.fori_loop(0, -(-n // NBUF), grp, 0)
        for b in range(NBUF):
            scatter_wait(b)


_SC_OUT = [jax.ShapeDtypeStruct((ACC_ROWS, DIN), jnp.float32)] * 2


def _core_out_copy(c, s, acc, out0, out1):
    @pl.when(c == 0)
    def _():
        pltpu.sync_copy(acc.at[pl.ds(s * RPT, RPT)],
                        out0.at[pl.ds(s * RPT, RPT)])

    @pl.when(c == 1)
    def _():
        pltpu.sync_copy(acc.at[pl.ds(s * RPT, RPT)],
                        out1.at[pl.ds(s * RPT, RPT)])


@functools.partial(
    pl.kernel,
    mesh=_sc_mesh,
    out_type=_SC_OUT,
    scratch_types=_SC_SCRATCH,
)
def _sc_segment_sum(table, srcs, dsts, zeros, out0, out1, src_v, dst_v,
                    rows_v, acc, gsem, ssem):
    c = lax.axis_index("c")
    s = lax.axis_index("s")
    w = c * NS + s
    pltpu.sync_copy(zeros.at[pl.ds(s * RPT, RPT)], acc.at[pl.ds(s * RPT, RPT)])
    plsc.subcore_barrier()
    _edge_pipeline(table, srcs, dsts, w, K, src_v, dst_v, rows_v, acc,
                   gsem, ssem)
    plsc.subcore_barrier()
    _core_out_copy(c, s, acc, out0, out1)


@functools.partial(
    pl.kernel,
    mesh=_sc_mesh,
    out_type=_SC_OUT,
    scratch_types=_SC_SCRATCH,
)
def _sc_segment_sum2(lo, hi, srcs, dsts, zeros, out0, out1, src_v, dst_v,
                     rows_v, acc, gsem, ssem):
    c = lax.axis_index("c")
    s = lax.axis_index("s")
    pltpu.sync_copy(zeros.at[pl.ds(s * RPT, RPT)], acc.at[pl.ds(s * RPT, RPT)])
    plsc.subcore_barrier()

    @pl.when(c == 0)
    def _():
        _edge_pipeline(lo, srcs, dsts, s, K2, src_v, dst_v, rows_v, acc,
                       gsem, ssem)

    @pl.when(c == 1)
    def _():
        _edge_pipeline(hi, srcs, dsts, s, K2, src_v, dst_v, rows_v, acc,
                       gsem, ssem)

    plsc.subcore_barrier()
    _core_out_copy(c, s, acc, out0, out1)


def _mlp1_body(x_ref, p0_ref, p1_ref, wa_ref, ba_ref, sc_ref, sh_ref, wb_ref,
               bb_ref, lo_ref, hi_ref):
    h = x_ref[...] + p0_ref[...] + p1_ref[...]
    h = jnp.dot(h, wa_ref[...], preferred_element_type=jnp.float32) + ba_ref[...]
    h = jnp.maximum(h * sc_ref[...] + sh_ref[...], 0.0)
    h = jnp.dot(h, wb_ref[...], preferred_element_type=jnp.float32) + bb_ref[...]
    h = jnp.maximum(h, 0.0)
    lo_ref[...] = h[:, :DIN]
    hi_ref[...] = h[:, DIN:]


def _mlp1(x, p0, p1, wa, ba, sc, sh, wb, bb):
    row = lambda i: (i, 0)
    full = lambda i: (0, 0)
    return pl.pallas_call(
        _mlp1_body,
        grid=(NBLK,),
        in_specs=[
            pl.BlockSpec((RB, DIN), row),
            pl.BlockSpec((RB, DIN), row),
            pl.BlockSpec((RB, DIN), row),
            pl.BlockSpec((DIN, F), full),
            pl.BlockSpec((1, F), full),
            pl.BlockSpec((1, F), full),
            pl.BlockSpec((1, F), full),
            pl.BlockSpec((F, F), full),
            pl.BlockSpec((1, F), full),
        ],
        out_specs=[pl.BlockSpec((RB, DIN), row), pl.BlockSpec((RB, DIN), row)],
        out_shape=[jax.ShapeDtypeStruct((N, DIN), jnp.float32)] * 2,
    )(x, p0, p1, wa, ba, sc, sh, wb, bb)


def _mlp2_body(lo_ref, hi_ref, q0_ref, q1_ref, bc_ref,
               wa_ref, ba_ref, sc_ref, sh_ref, wb_ref, bb_ref, w3a_ref, b3a_ref,
               w3b_ref, b3b_ref, out_ref, acc_sum, acc_max, acc_cnt):
    i = pl.program_id(0)
    hlo = lo_ref[...] + q0_ref[...]
    hhi = hi_ref[...] + q1_ref[...]
    h = jnp.concatenate([hlo, hhi], axis=1)
    h = jnp.dot(h, wa_ref[...], preferred_element_type=jnp.float32) + ba_ref[...]
    h = jnp.maximum(h * sc_ref[...] + sh_ref[...], 0.0)
    h = jnp.dot(h, wb_ref[...], preferred_element_type=jnp.float32) + bb_ref[...]
    h = jnp.maximum(h, 0.0)

    bc = bc_ref[...]
    gids = lax.broadcasted_iota(jnp.int32, (RB, G), 1)
    onehot = (gids == bc).astype(jnp.float32)
    dn = (((0,), (0,)), ((), ()))
    blk_sum = lax.dot_general(onehot, h, dn, preferred_element_type=jnp.float32)
    blk_cnt = lax.dot_general(onehot, jnp.ones_like(h), dn,
                              preferred_element_type=jnp.float32)
    neg = jnp.float32(-jnp.inf)
    blk_max = jnp.concatenate(
        [jnp.max(jnp.where(bc == g, h, neg), axis=0, keepdims=True)
         for g in range(G)], axis=0)

    @pl.when(i == 0)
    def _():
        acc_sum[...] = blk_sum
        acc_cnt[...] = blk_cnt
        acc_max[...] = blk_max

    @pl.when(i > 0)
    def _():
        acc_sum[...] += blk_sum
        acc_cnt[...] += blk_cnt
        acc_max[...] = jnp.maximum(acc_max[...], blk_max)

    @pl.when(i == NBLK - 1)
    def _():
        mean = acc_sum[...] / jnp.maximum(acc_cnt[...], 1.0)
        r = jnp.concatenate([mean, acc_max[...]], axis=1)
        t = jnp.maximum(
            jnp.dot(r, w3a_ref[...], preferred_element_type=jnp.float32)
            + b3a_ref[...], 0.0)
        out_ref[...] = (jnp.dot(t, w3b_ref[...], preferred_element_type=jnp.float32)
                        + b3b_ref[...])


def _mlp2(lo, hi, q0, q1, bc, wa, ba, sc, sh, wb, bb, w3a, b3a,
          w3b, b3b):
    row = lambda i: (i, 0)
    full = lambda i: (0, 0)
    return pl.pallas_call(
        _mlp2_body,
        grid=(NBLK,),
        in_specs=[
            pl.BlockSpec((RB, DIN), row),
            pl.BlockSpec((RB, DIN), row),
            pl.BlockSpec((RB, DIN), row),
            pl.BlockSpec((RB, DIN), row),
            pl.BlockSpec((RB, 1), row),
            pl.BlockSpec((F, F), full),
            pl.BlockSpec((1, F), full),
            pl.BlockSpec((1, F), full),
            pl.BlockSpec((1, F), full),
            pl.BlockSpec((F, F), full),
            pl.BlockSpec((1, F), full),
            pl.BlockSpec((2 * F, 4 * F), full),
            pl.BlockSpec((1, 4 * F), full),
            pl.BlockSpec((4 * F, 2), full),
            pl.BlockSpec((1, 2), full),
        ],
        out_specs=pl.BlockSpec((G, 2), full),
        out_shape=jax.ShapeDtypeStruct((G, 2), jnp.float32),
        scratch_shapes=[
            pltpu.VMEM((G, F), jnp.float32),
            pltpu.VMEM((G, F), jnp.float32),
            pltpu.VMEM((G, F), jnp.float32),
        ],
    )(lo, hi, q0, q1, bc, wa, ba, sc, sh, wb, bb, w3a, b3a, w3b, b3b)


def kernel(x, edge_index, batch, W1a, b1a, g1, be1, rm1, rv1, W1b, b1b,
           W2a, b2a, g2, be2, rm2, rv2, W2b, b2b, W3a, b3a, W3b, b3b):
    src = edge_index[0]
    dst = edge_index[1]
    pad = EPAD - E
    pad_ids = jnp.arange(pad, dtype=jnp.int32)
    src_p = jnp.concatenate([src, pad_ids % N]).reshape(NW, K, CH)
    dst_p = jnp.concatenate(
        [dst, N + pad_ids % (ACC_ROWS - N)]).reshape(NW, K, CH)
    zeros = jnp.zeros((ACC_ROWS, DIN), jnp.float32)

    inv1 = g1 / jnp.sqrt(rv1 + 1e-5)
    sc1 = inv1.reshape(1, F)
    sh1 = (be1 - rm1 * inv1).reshape(1, F)
    inv2 = g2 / jnp.sqrt(rv2 + 1e-5)
    sc2 = inv2.reshape(1, F)
    sh2 = (be2 - rm2 * inv2).reshape(1, F)

    pad2_ids = jnp.arange(EPAD2 - E, dtype=jnp.int32)
    src2 = jnp.concatenate([src, pad2_ids % N]).reshape(NS, K2, CH)
    dst2 = jnp.concatenate(
        [dst, N + pad2_ids % (ACC_ROWS - N)]).reshape(NS, K2, CH)

    p1a, p1b = _sc_segment_sum(x, src_p, dst_p, zeros)
    lo, hi = _mlp1(x, p1a, p1b, W1a, b1a.reshape(1, F),
                   sc1, sh1, W1b, b1b.reshape(1, F))

    agg_lo, agg_hi = _sc_segment_sum2(lo, hi, src2, dst2, zeros)

    bc = batch.reshape(N, 1)
    out = _mlp2(lo, hi, agg_lo, agg_hi, bc,
                W2a, b2a.reshape(1, F), sc2, sh2,
                W2b, b2b.reshape(1, F), W3a, b3a.reshape(1, 4 * F),
                W3b, b3b.reshape(1, 2))
    return out
